# baseline (device time: 253761 ns/iter reference)
import jax
import jax.numpy as jnp
from jax import lax
from jax.experimental import pallas as pl
from jax.experimental.pallas import tpu as pltpu

CHUNK_ROWS = [512] * 6 + [384, 256, 160, 96, 64, 32, 32]
N_CHUNK = len(CHUNK_ROWS)
L_ROWS = 512
N_LOCAL = 16


def kernel(x):
    m, n_full = x.shape
    n = n_full // 2
    half = m // 2
    offs = [sum(CHUNK_ROWS[:c]) for c in range(N_CHUNK)]

    def body(x_hbm, out_hbm, y_send, y_recv, f_send, f_recv, vbuf, vsem):
        mx = lax.axis_index("x")
        my = lax.axis_index("y")

        barrier = pltpu.get_barrier_semaphore()
        pl.semaphore_signal(barrier, inc=1, device_id=(mx, 1 - my),
                            device_id_type=pl.DeviceIdType.MESH)
        pl.semaphore_signal(barrier, inc=1, device_id=(1 - mx, my),
                            device_id_type=pl.DeviceIdType.MESH)
        pl.semaphore_wait(barrier, 2)

        y_rdmas = []
        for c in range(N_CHUNK):
            rows = CHUNK_ROWS[c]
            y_rdma = pltpu.make_async_remote_copy(
                src_ref=x_hbm.at[pl.ds(mx * half + offs[c], rows),
                                 pl.ds((1 - my) * n, n)],
                dst_ref=out_hbm.at[pl.ds(my * m + mx * half + offs[c], rows), :],
                send_sem=y_send.at[c],
                recv_sem=y_recv.at[c],
                device_id=(mx, 1 - my),
                device_id_type=pl.DeviceIdType.MESH,
            )
            y_rdma.start()
            y_rdmas.append(y_rdma)

        fwds = []
        stores = [None, None]
        lc = 0

        def do_local_chunk():
            nonlocal lc
            slot = lc % 2
            ld = pltpu.make_async_copy(
                x_hbm.at[pl.ds(lc * L_ROWS, L_ROWS), pl.ds(my * n, n)],
                vbuf.at[slot],
                vsem.at[slot],
            )
            if stores[slot] is not None:
                stores[slot].wait()
            ld.start()
            ld.wait()
            st = pltpu.make_async_copy(
                vbuf.at[slot],
                out_hbm.at[pl.ds(my * m + lc * L_ROWS, L_ROWS), :],
                vsem.at[2 + slot],
            )
            st.start()
            stores[slot] = st
            lc += 1

        for c in range(N_CHUNK):
            rows = CHUNK_ROWS[c]
            y_rdmas[c].wait_recv()
            r = (1 - my) * m + mx * half + offs[c]
            fwd = pltpu.make_async_remote_copy(
                src_ref=out_hbm.at[pl.ds(r, rows), :],
                dst_ref=out_hbm.at[pl.ds(r, rows), :],
                send_sem=f_send.at[c],
                recv_sem=f_recv.at[c],
                device_id=(1 - mx, my),
                device_id_type=pl.DeviceIdType.MESH,
            )
            fwd.start()
            fwds.append(fwd)
            if c < 8:
                do_local_chunk()
                do_local_chunk()

        for c in range(N_CHUNK):
            y_rdmas[c].wait_send()
            fwds[c].wait()
        stores[0].wait()
        stores[1].wait()

    return pl.pallas_call(
        body,
        out_shape=jax.ShapeDtypeStruct((2 * m, n), x.dtype),
        in_specs=[pl.BlockSpec(memory_space=pl.ANY)],
        out_specs=pl.BlockSpec(memory_space=pl.ANY),
        scratch_shapes=[
            pltpu.SemaphoreType.DMA((N_CHUNK,)),
            pltpu.SemaphoreType.DMA((N_CHUNK,)),
            pltpu.SemaphoreType.DMA((N_CHUNK,)),
            pltpu.SemaphoreType.DMA((N_CHUNK,)),
            pltpu.VMEM((2, L_ROWS, n), x.dtype),
            pltpu.SemaphoreType.DMA((4,)),
        ],
        compiler_params=pltpu.CompilerParams(collective_id=0),
    )(x)


# device time: 237646 ns/iter; 1.0678x vs baseline; 1.0678x over previous
import jax
import jax.numpy as jnp
from jax import lax
from jax.experimental import pallas as pl
from jax.experimental.pallas import tpu as pltpu

CHUNK_ROWS = [128] * 32
N_CHUNK = len(CHUNK_ROWS)
L_ROWS = 512


def kernel(x):
    m, n_full = x.shape
    n = n_full // 2
    half = m // 2
    offs = [sum(CHUNK_ROWS[:c]) for c in range(N_CHUNK)]

    def body(x_hbm, out_hbm, y_send, y_recv, f_send, f_recv, vbuf, vsem):
        mx = lax.axis_index("x")
        my = lax.axis_index("y")

        barrier = pltpu.get_barrier_semaphore()
        pl.semaphore_signal(barrier, inc=1, device_id=(mx, 1 - my),
                            device_id_type=pl.DeviceIdType.MESH)
        pl.semaphore_signal(barrier, inc=1, device_id=(1 - mx, my),
                            device_id_type=pl.DeviceIdType.MESH)
        pl.semaphore_wait(barrier, 2)

        y_rdmas = []
        for c in range(N_CHUNK):
            rows = CHUNK_ROWS[c]
            y_rdma = pltpu.make_async_remote_copy(
                src_ref=x_hbm.at[pl.ds(mx * half + offs[c], rows),
                                 pl.ds((1 - my) * n, n)],
                dst_ref=out_hbm.at[pl.ds(my * m + mx * half + offs[c], rows), :],
                send_sem=y_send.at[c],
                recv_sem=y_recv.at[c],
                device_id=(mx, 1 - my),
                device_id_type=pl.DeviceIdType.MESH,
            )
            y_rdma.start()
            y_rdmas.append(y_rdma)

        fwds = []
        stores = [None, None]
        lc = 0

        def do_local_chunk():
            nonlocal lc
            slot = lc % 2
            ld = pltpu.make_async_copy(
                x_hbm.at[pl.ds(lc * L_ROWS, L_ROWS), pl.ds(my * n, n)],
                vbuf.at[slot],
                vsem.at[slot],
            )
            if stores[slot] is not None:
                stores[slot].wait()
            ld.start()
            ld.wait()
            st = pltpu.make_async_copy(
                vbuf.at[slot],
                out_hbm.at[pl.ds(my * m + lc * L_ROWS, L_ROWS), :],
                vsem.at[2 + slot],
            )
            st.start()
            stores[slot] = st
            lc += 1

        for c in range(N_CHUNK):
            rows = CHUNK_ROWS[c]
            y_rdmas[c].wait_recv()
            r = (1 - my) * m + mx * half + offs[c]
            fwd = pltpu.make_async_remote_copy(
                src_ref=out_hbm.at[pl.ds(r, rows), :],
                dst_ref=out_hbm.at[pl.ds(r, rows), :],
                send_sem=f_send.at[c],
                recv_sem=f_recv.at[c],
                device_id=(1 - mx, my),
                device_id_type=pl.DeviceIdType.MESH,
            )
            fwd.start()
            fwds.append(fwd)
            if c < 16:
                do_local_chunk()

        for c in range(N_CHUNK):
            y_rdmas[c].wait_send()
            fwds[c].wait()
        stores[0].wait()
        stores[1].wait()

    return pl.pallas_call(
        body,
        out_shape=jax.ShapeDtypeStruct((2 * m, n), x.dtype),
        in_specs=[pl.BlockSpec(memory_space=pl.ANY)],
        out_specs=pl.BlockSpec(memory_space=pl.ANY),
        scratch_shapes=[
            pltpu.SemaphoreType.DMA((N_CHUNK,)),
            pltpu.SemaphoreType.DMA((N_CHUNK,)),
            pltpu.SemaphoreType.DMA((N_CHUNK,)),
            pltpu.SemaphoreType.DMA((N_CHUNK,)),
            pltpu.VMEM((2, L_ROWS, n), x.dtype),
            pltpu.SemaphoreType.DMA((4,)),
        ],
        compiler_params=pltpu.CompilerParams(collective_id=0),
    )(x)


# device time: 236003 ns/iter; 1.0752x vs baseline; 1.0070x over previous
import jax
import jax.numpy as jnp
from jax import lax
from jax.experimental import pallas as pl
from jax.experimental.pallas import tpu as pltpu

CHUNK_ROWS = [64] * 64
N_CHUNK = len(CHUNK_ROWS)
L_ROWS = 512


def kernel(x):
    m, n_full = x.shape
    n = n_full // 2
    half = m // 2
    offs = [sum(CHUNK_ROWS[:c]) for c in range(N_CHUNK)]

    def body(x_hbm, out_hbm, y_send, y_recv, f_send, f_recv, vbuf, vsem):
        mx = lax.axis_index("x")
        my = lax.axis_index("y")

        barrier = pltpu.get_barrier_semaphore()
        pl.semaphore_signal(barrier, inc=1, device_id=(mx, 1 - my),
                            device_id_type=pl.DeviceIdType.MESH)
        pl.semaphore_signal(barrier, inc=1, device_id=(1 - mx, my),
                            device_id_type=pl.DeviceIdType.MESH)
        pl.semaphore_wait(barrier, 2)

        y_rdmas = []
        for c in range(N_CHUNK):
            rows = CHUNK_ROWS[c]
            y_rdma = pltpu.make_async_remote_copy(
                src_ref=x_hbm.at[pl.ds(mx * half + offs[c], rows),
                                 pl.ds((1 - my) * n, n)],
                dst_ref=out_hbm.at[pl.ds(my * m + mx * half + offs[c], rows), :],
                send_sem=y_send.at[c],
                recv_sem=y_recv.at[c],
                device_id=(mx, 1 - my),
                device_id_type=pl.DeviceIdType.MESH,
            )
            y_rdma.start()
            y_rdmas.append(y_rdma)

        fwds = []
        stores = [None, None]
        lc = 0

        def do_local_chunk():
            nonlocal lc
            slot = lc % 2
            ld = pltpu.make_async_copy(
                x_hbm.at[pl.ds(lc * L_ROWS, L_ROWS), pl.ds(my * n, n)],
                vbuf.at[slot],
                vsem.at[slot],
            )
            if stores[slot] is not None:
                stores[slot].wait()
            ld.start()
            ld.wait()
            st = pltpu.make_async_copy(
                vbuf.at[slot],
                out_hbm.at[pl.ds(my * m + lc * L_ROWS, L_ROWS), :],
                vsem.at[2 + slot],
            )
            st.start()
            stores[slot] = st
            lc += 1

        for c in range(N_CHUNK):
            rows = CHUNK_ROWS[c]
            y_rdmas[c].wait_recv()
            r = (1 - my) * m + mx * half + offs[c]
            fwd = pltpu.make_async_remote_copy(
                src_ref=out_hbm.at[pl.ds(r, rows), :],
                dst_ref=out_hbm.at[pl.ds(r, rows), :],
                send_sem=f_send.at[c],
                recv_sem=f_recv.at[c],
                device_id=(1 - mx, my),
                device_id_type=pl.DeviceIdType.MESH,
            )
            fwd.start()
            fwds.append(fwd)
            if c % 4 == 0:
                do_local_chunk()

        for c in range(N_CHUNK):
            y_rdmas[c].wait_send()
            fwds[c].wait()
        stores[0].wait()
        stores[1].wait()

    return pl.pallas_call(
        body,
        out_shape=jax.ShapeDtypeStruct((2 * m, n), x.dtype),
        in_specs=[pl.BlockSpec(memory_space=pl.ANY)],
        out_specs=pl.BlockSpec(memory_space=pl.ANY),
        scratch_shapes=[
            pltpu.SemaphoreType.DMA((N_CHUNK,)),
            pltpu.SemaphoreType.DMA((N_CHUNK,)),
            pltpu.SemaphoreType.DMA((N_CHUNK,)),
            pltpu.SemaphoreType.DMA((N_CHUNK,)),
            pltpu.VMEM((2, L_ROWS, n), x.dtype),
            pltpu.SemaphoreType.DMA((4,)),
        ],
        compiler_params=pltpu.CompilerParams(collective_id=0),
    )(x)
